# Initial kernel scaffold; baseline (speedup 1.0000x reference)
import jax
import jax.numpy as jnp
from jax.experimental import pallas as pl


def kernel(seq_S_u, seq_P_u, T_delta_u, contract_table, time_K_table, time_V_table, pop_Q_table, pop_K_table, pop_V_table):
    raise NotImplementedError("write your pallas kernel here")


# speedup vs baseline: 3.0685x; 3.0685x over previous
import jax
import jax.numpy as jnp
from jax import lax
from jax.experimental import pallas as pl
from jax.experimental.pallas import tpu as pltpu
from jax.experimental.pallas import tpu_sc as plsc

NC = 2
NS = 16
NW = NC * NS
CHUNK = 128
D = 64
LANES = 16


def _small_gather(B, tables_shape, n_tables):
    V = tables_shape[0]
    b_per_w = B // NW
    n_chunks = b_per_w // CHUNK
    assert b_per_w * NW == B and n_chunks * CHUNK == b_per_w

    mesh = plsc.VectorSubcoreMesh(core_axis_name="c", subcore_axis_name="s")
    out_type = [jax.ShapeDtypeStruct((B, D), jnp.float32) for _ in range(n_tables)]
    scratch = [pltpu.VMEM((V, D), jnp.float32) for _ in range(n_tables)]
    scratch += [pltpu.VMEM((CHUNK,), jnp.int32)]
    scratch += [pltpu.VMEM((CHUNK, D), jnp.float32) for _ in range(n_tables)]
    scratch += [pltpu.SemaphoreType.DMA]

    def body(idx_hbm, *refs):
        tables = refs[:n_tables]
        outs = refs[n_tables:2 * n_tables]
        tab_v = refs[2 * n_tables:3 * n_tables]
        idx_v = refs[3 * n_tables]
        rows = refs[3 * n_tables + 1:3 * n_tables + 1 + n_tables]
        sem = refs[-1]

        wid = lax.axis_index("s") * NC + lax.axis_index("c")
        base = wid * b_per_w

        for t in range(n_tables):
            pltpu.sync_copy(tables[t], tab_v[t])

        def chunk_body(j, carry):
            off = base + j * CHUNK
            pltpu.sync_copy(idx_hbm.at[pl.ds(off, CHUNK)], idx_v)

            def group_body(g, c2):
                iv = idx_v[pl.ds(g * LANES, LANES)]
                for l in range(LANES):
                    s = iv[l]
                    r = g * LANES + l
                    for t in range(n_tables):
                        for c in range(D // LANES):
                            sl = pl.ds(c * LANES, LANES)
                            rows[t][r, sl] = tab_v[t][s, sl]
                return c2

            lax.fori_loop(0, CHUNK // LANES, group_body, 0)
            for t in range(n_tables):
                pltpu.sync_copy(rows[t], outs[t].at[pl.ds(off, CHUNK)])
            return carry

        lax.fori_loop(0, n_chunks, chunk_body, 0)

    return pl.kernel(body, out_type=out_type, mesh=mesh, scratch_types=scratch)


def _big_gather(B):
    b_per_w = B // NW
    n_chunks = b_per_w // CHUNK
    assert b_per_w * NW == B and n_chunks * CHUNK == b_per_w

    mesh = plsc.VectorSubcoreMesh(core_axis_name="c", subcore_axis_name="s")
    out_type = jax.ShapeDtypeStruct((B, D), jnp.float32)
    scratch = [
        pltpu.VMEM((CHUNK,), jnp.int32),
        pltpu.VMEM((CHUNK, 2 * D), jnp.float32),
        pltpu.VMEM((CHUNK, D), jnp.float32),
        pltpu.SemaphoreType.DMA,
    ]

    def body(table_hbm, idx_hbm, out, idx_v, wide, rows, sem):
        wid = lax.axis_index("s") * NC + lax.axis_index("c")
        base = wid * b_per_w

        def chunk_body(j, carry):
            off = base + j * CHUNK
            pltpu.sync_copy(idx_hbm.at[pl.ds(off, CHUNK)], idx_v)
            pltpu.async_copy(table_hbm.at[idx_v], wide, sem).wait()

            def row_body(r, c2):
                for c in range(D // LANES):
                    rows[r, pl.ds(c * LANES, LANES)] = wide[r, pl.ds(c * LANES, LANES)]
                return c2

            lax.fori_loop(0, CHUNK, row_body, 0)
            pltpu.sync_copy(rows, out.at[pl.ds(off, CHUNK)])
            return carry

        lax.fori_loop(0, n_chunks, chunk_body, 0)

    return pl.kernel(body, out_type=out_type, mesh=mesh, scratch_types=scratch)


def kernel(seq_S_u, seq_P_u, T_delta_u, contract_table, time_K_table,
           time_V_table, pop_Q_table, pop_K_table, pop_V_table):
    B, L = seq_S_u.shape

    idx_S = seq_S_u.astype(jnp.int32).reshape(-1)
    idx_T = T_delta_u.astype(jnp.int32).reshape(-1)
    idx_P = seq_P_u.astype(jnp.int32).reshape(-1)

    contract_wide = jnp.pad(contract_table, ((0, 0), (0, D)))

    E = _big_gather(B * L)(contract_wide, idx_S)
    T_K, T_V = _small_gather(B * L * L, time_K_table.shape, 2)(
        idx_T, time_K_table, time_V_table)
    P_Q, P_K, P_V = _small_gather(B * L, pop_Q_table.shape, 3)(
        idx_P, pop_Q_table, pop_K_table, pop_V_table)

    return (
        E.reshape(B, L, D),
        T_K.reshape(B, L, L, D),
        T_V.reshape(B, L, L, D),
        P_Q.reshape(B, L, D),
        P_K.reshape(B, L, D),
        P_V.reshape(B, L, D),
    )


# speedup vs baseline: 4.0949x; 1.3345x over previous
import jax
import jax.numpy as jnp
from jax import lax
from jax.experimental import pallas as pl
from jax.experimental.pallas import tpu as pltpu
from jax.experimental.pallas import tpu_sc as plsc

NC = 2
NS = 16
NW = NC * NS
LANES = 16
B = 1024
L = 20
D = 64
NBW = B // NW
H = 4
NSL = D // LANES


def _copy_row(dst, dst_idx, src, src_row):
    for c in range(NSL):
        sl = pl.ds(c * LANES, LANES)
        dst[dst_idx + (sl,)] = src[src_row, sl]


def _time_gather():
    rpc = H * L
    cpb = L // H
    n_pairs = NBW * cpb // 2
    slab = NBW * L * L

    mesh = plsc.VectorSubcoreMesh(core_axis_name="c", subcore_axis_name="s")
    out_type = [jax.ShapeDtypeStruct((B, L, L, D), jnp.float32) for _ in range(2)]
    scratch = [pltpu.VMEM((256, D), jnp.float32) for _ in range(2)]
    scratch += [pltpu.VMEM((slab,), jnp.int32)]
    scratch += [pltpu.VMEM((H, L, D), jnp.float32) for _ in range(4)]
    scratch += [pltpu.SemaphoreType.DMA for _ in range(4)]

    def body(idx_hbm, tk_hbm, tv_hbm, ok_hbm, ov_hbm, *refs):
        tabs = refs[0:2]
        idx_v = refs[2]
        rows = (refs[3:5], refs[5:7])
        sems = (refs[7:9], refs[9:11])
        outs = (ok_hbm, ov_hbm)

        wid = lax.axis_index("s") * NC + lax.axis_index("c")
        b0 = wid * NBW

        pltpu.sync_copy(tk_hbm, tabs[0])
        pltpu.sync_copy(tv_hbm, tabs[1])
        pltpu.sync_copy(idx_hbm.at[pl.ds(wid * slab, slab)], idx_v)

        def pair_body(p, carry):
            for i in range(2):
                k = p * 2 + i
                b = b0 + k // cpb
                h = lax.rem(k, cpb)

                @pl.when(p > 0)
                def _wait():
                    for t in range(2):
                        pltpu.make_async_copy(
                            outs[t].at[b0, pl.ds(0, H)], rows[t][i], sems[t][i]
                        ).wait()

                for g in range(rpc // LANES):
                    iv = idx_v[pl.ds(k * rpc + g * LANES, LANES)]
                    for l in range(LANES):
                        r = g * LANES + l
                        l1, l2 = r // L, r % L
                        s = iv[l]
                        for t in range(2):
                            _copy_row(rows[t][i], (l1, l2), tabs[t], s)

                for t in range(2):
                    pltpu.async_copy(
                        rows[t][i], outs[t].at[b, pl.ds(h * H, H)], sems[t][i]
                    )
            return carry

        lax.fori_loop(0, n_pairs, pair_body, 0)
        for t in range(2):
            for i in range(2):
                pltpu.make_async_copy(
                    outs[t].at[b0, pl.ds(0, H)], rows[t][i], sems[t][i]
                ).wait()

    return pl.kernel(body, out_type=out_type, mesh=mesh, scratch_types=scratch)


def _pop_gather(V):
    rpc = H * L
    n_pairs = NBW // H // 2
    slab = NBW * L

    mesh = plsc.VectorSubcoreMesh(core_axis_name="c", subcore_axis_name="s")
    out_type = [jax.ShapeDtypeStruct((B, L, D), jnp.float32) for _ in range(3)]
    scratch = [pltpu.VMEM((V, D), jnp.float32) for _ in range(3)]
    scratch += [pltpu.VMEM((slab,), jnp.int32)]
    scratch += [pltpu.VMEM((H, L, D), jnp.float32) for _ in range(6)]
    scratch += [pltpu.SemaphoreType.DMA for _ in range(6)]

    def body(idx_hbm, *refs):
        tabs_hbm = refs[0:3]
        outs = refs[3:6]
        tabs = refs[6:9]
        idx_v = refs[9]
        rows = (refs[10:12], refs[12:14], refs[14:16])
        sems = (refs[16:18], refs[18:20], refs[20:22])

        wid = lax.axis_index("s") * NC + lax.axis_index("c")
        b0 = wid * NBW

        for t in range(3):
            pltpu.sync_copy(tabs_hbm[t], tabs[t])
        pltpu.sync_copy(idx_hbm.at[pl.ds(wid * slab, slab)], idx_v)

        def pair_body(p, carry):
            for i in range(2):
                k = p * 2 + i
                bb = b0 + k * H

                @pl.when(p > 0)
                def _wait():
                    for t in range(3):
                        pltpu.make_async_copy(
                            outs[t].at[pl.ds(b0, H)], rows[t][i], sems[t][i]
                        ).wait()

                for g in range(rpc // LANES):
                    iv = idx_v[pl.ds(k * rpc + g * LANES, LANES)]
                    for l in range(LANES):
                        r = g * LANES + l
                        ib, ll = r // L, r % L
                        s = iv[l]
                        for t in range(3):
                            _copy_row(rows[t][i], (ib, ll), tabs[t], s)

                for t in range(3):
                    pltpu.async_copy(
                        rows[t][i], outs[t].at[pl.ds(bb, H)], sems[t][i]
                    )
            return carry

        lax.fori_loop(0, n_pairs, pair_body, 0)
        for t in range(3):
            for i in range(2):
                pltpu.make_async_copy(
                    outs[t].at[pl.ds(b0, H)], rows[t][i], sems[t][i]
                ).wait()

    return pl.kernel(body, out_type=out_type, mesh=mesh, scratch_types=scratch)


def _contract_gather():
    rpc = H * L
    n_chunks = NBW // H
    slab = NBW * L

    mesh = plsc.VectorSubcoreMesh(core_axis_name="c", subcore_axis_name="s")
    out_type = jax.ShapeDtypeStruct((B, L, D), jnp.float32)
    scratch = [pltpu.VMEM((slab,), jnp.int32)]
    scratch += [pltpu.VMEM((rpc, 2 * D), jnp.float32) for _ in range(2)]
    scratch += [pltpu.VMEM((H, L, D), jnp.float32) for _ in range(2)]
    scratch += [pltpu.SemaphoreType.DMA for _ in range(4)]

    def body(table_hbm, idx_hbm, out, idx_v, w0, w1, r0, r1, gs0, gs1, os0, os1):
        wide = (w0, w1)
        rows = (r0, r1)
        gsems = (gs0, gs1)
        osems = (os0, os1)

        wid = lax.axis_index("s") * NC + lax.axis_index("c")
        b0 = wid * NBW

        pltpu.sync_copy(idx_hbm.at[pl.ds(wid * slab, slab)], idx_v)
        pltpu.async_copy(table_hbm.at[idx_v.at[pl.ds(0, rpc)]], wide[0], gsems[0])

        def pair_body(p, carry):
            for i in range(2):
                k = p * 2 + i

                @pl.when(k + 1 < n_chunks)
                def _prefetch():
                    pltpu.async_copy(
                        table_hbm.at[idx_v.at[pl.ds((k + 1) * rpc, rpc)]],
                        wide[1 - i], gsems[1 - i],
                    )

                pltpu.make_async_copy(
                    table_hbm.at[idx_v.at[pl.ds(0, rpc)]], wide[i], gsems[i]
                ).wait()

                @pl.when(p > 0)
                def _wait_out():
                    pltpu.make_async_copy(
                        out.at[pl.ds(b0, H)], rows[i], osems[i]
                    ).wait()

                for ib in range(H):
                    for ll in range(L):
                        _copy_row(rows[i], (ib, ll), wide[i], ib * L + ll)

                pltpu.async_copy(rows[i], out.at[pl.ds(b0 + k * H, H)], osems[i])
            return carry

        lax.fori_loop(0, n_chunks // 2, pair_body, 0)
        for i in range(2):
            pltpu.make_async_copy(out.at[pl.ds(b0, H)], rows[i], osems[i]).wait()

    return pl.kernel(body, out_type=out_type, mesh=mesh, scratch_types=scratch)


def kernel(seq_S_u, seq_P_u, T_delta_u, contract_table, time_K_table,
           time_V_table, pop_Q_table, pop_K_table, pop_V_table):
    idx_S = seq_S_u.astype(jnp.int32).reshape(-1)
    idx_T = T_delta_u.astype(jnp.int32).reshape(-1)
    idx_P = seq_P_u.astype(jnp.int32).reshape(-1)

    contract_wide = jnp.pad(contract_table, ((0, 0), (0, D)))

    E = _contract_gather()(contract_wide, idx_S)
    T_K, T_V = _time_gather()(idx_T, time_K_table, time_V_table)
    P_Q, P_K, P_V = _pop_gather(pop_Q_table.shape[0])(
        idx_P, pop_Q_table, pop_K_table, pop_V_table)

    return (E, T_K, T_V, P_Q, P_K, P_V)
